# Initial kernel scaffold; baseline (speedup 1.0000x reference)
import jax
import jax.numpy as jnp
from jax.experimental import pallas as pl


def kernel(node_feat, edge_index, batch, W, b):
    raise NotImplementedError("write your pallas kernel here")


# speedup vs baseline: 16.0009x; 16.0009x over previous
import functools

import jax
import jax.numpy as jnp
from jax import lax
from jax.experimental import pallas as pl
from jax.experimental.pallas import tpu as pltpu
from jax.experimental.pallas import tpu_sc as plsc

N = 10000
D = 128
G = 64

NC = 2
NS = 16
NT = NC * NS
CHUNK = 128

NPAD = 10240


def _sc_mesh():
    return plsc.VectorSubcoreMesh(
        core_axis_name="c", subcore_axis_name="s", num_cores=NC, num_subcores=NS
    )


def _make_deg_kernel(ec):
    @functools.partial(
        pl.kernel,
        out_type=jax.ShapeDtypeStruct((NC, NPAD, 16), jnp.float32),
        mesh=_sc_mesh(),
        scratch_types=[
            pltpu.VMEM((CHUNK,), jnp.int32),
            pltpu.VMEM((CHUNK, 16), jnp.float32),
            pltpu.VMEM_SHARED((NPAD, 16), jnp.float32),
            pltpu.SemaphoreType.DMA,
        ],
    )
    def deg_kernel(dst_hbm, zeros_hbm, out_hbm, dstv, ones_v, deg_sh, sem):
        c = lax.axis_index("c")
        s = lax.axis_index("s")
        wid = c * NS + s

        def fill(i, _):
            ones_v[i, :] = jnp.ones((16,), jnp.float32)
            return _

        lax.fori_loop(0, CHUNK, fill, None)

        @pl.when(s == 0)
        def _():
            pltpu.sync_copy(zeros_hbm, deg_sh)

        plsc.subcore_barrier()

        def step(j, _):
            pltpu.sync_copy(dst_hbm.at[wid, j], dstv)
            pltpu.sync_copy(ones_v, deg_sh.at[dstv], add=True)
            return _

        lax.fori_loop(0, ec, step, None)
        plsc.subcore_barrier()

        @pl.when(s == 0)
        def _():
            pltpu.sync_copy(deg_sh, out_hbm.at[c])

    return deg_kernel


def _make_msg_kernel(ec):
    @functools.partial(
        pl.kernel,
        out_type=jax.ShapeDtypeStruct((NC, NPAD, D), jnp.float32),
        mesh=_sc_mesh(),
        scratch_types=[
            pltpu.VMEM((CHUNK,), jnp.int32),
            pltpu.VMEM((CHUNK,), jnp.int32),
            pltpu.VMEM((CHUNK, D), jnp.float32),
            pltpu.VMEM_SHARED((NPAD, D), jnp.float32),
            pltpu.SemaphoreType.DMA,
        ],
    )
    def msg_kernel(src_hbm, dst_hbm, y_hbm, zeros_hbm, out_hbm, srcv, dstv,
                   rows, acc_sh, sem):
        c = lax.axis_index("c")
        s = lax.axis_index("s")
        wid = c * NS + s

        @pl.when(s == 0)
        def _():
            pltpu.sync_copy(zeros_hbm, acc_sh)

        plsc.subcore_barrier()

        def step(j, _):
            pltpu.sync_copy(src_hbm.at[wid, j], srcv)
            pltpu.sync_copy(dst_hbm.at[wid, j], dstv)
            pltpu.async_copy(y_hbm.at[srcv], rows, sem).wait()
            pltpu.sync_copy(rows, acc_sh.at[dstv], add=True)
            return _

        lax.fori_loop(0, ec, step, None)
        plsc.subcore_barrier()

        @pl.when(s == 0)
        def _():
            pltpu.sync_copy(acc_sh, out_hbm.at[c])

    return msg_kernel


def _scale_matmul_kernel(x_ref, w_ref, deg_ref, y_ref):
    xw = jnp.dot(x_ref[...], w_ref[...], preferred_element_type=jnp.float32)
    d = deg_ref[0] + deg_ref[1]
    dinv = lax.rsqrt(d[:, 0:1] + 1.0)
    y_ref[...] = xw * dinv


def _finish_kernel(nblk, br, p_ref, y_ref, deg_ref, b_ref, batch_ref, out_ref,
                   cnt_ref):
    i = pl.program_id(0)

    @pl.when(i == 0)
    def _():
        out_ref[...] = jnp.zeros((G, D), jnp.float32)
        cnt_ref[...] = jnp.zeros((G, D), jnp.float32)

    d = deg_ref[0] + deg_ref[1]
    dinv = lax.rsqrt(d[:, 0:1] + 1.0)
    z = (p_ref[0] + p_ref[1] + y_ref[...]) * dinv + b_ref[...]
    z = jnp.maximum(z, 0.0)
    oh = (batch_ref[...] == lax.broadcasted_iota(jnp.int32, (br, G), 1))
    oh = oh.astype(jnp.float32)
    out_ref[...] += lax.dot_general(
        oh, z, (((0,), (0,)), ((), ())), preferred_element_type=jnp.float32
    )
    cnt_ref[...] += lax.dot_general(
        oh, jnp.ones((br, D), jnp.float32), (((0,), (0,)), ((), ())),
        preferred_element_type=jnp.float32,
    )

    @pl.when(i == nblk - 1)
    def _():
        out_ref[...] = out_ref[...] / jnp.maximum(cnt_ref[...], 1.0)


def kernel(node_feat, edge_index, batch, W, b):
    n, d_in = node_feat.shape
    e = edge_index.shape[1]
    ept = pl.cdiv(pl.cdiv(e, NT), CHUNK) * CHUNK
    ec = ept // CHUNK

    pad_e = NT * ept - e
    src_t = jnp.concatenate(
        [edge_index[0], jnp.full((pad_e,), N, jnp.int32)]
    ).reshape(NT, ec, CHUNK)
    dst_t = jnp.concatenate(
        [edge_index[1], jnp.full((pad_e,), N, jnp.int32)]
    ).reshape(NT, ec, CHUNK)
    x_pad = jnp.concatenate(
        [node_feat, jnp.zeros((NPAD - n, d_in), jnp.float32)]
    )
    batch_col = jnp.concatenate(
        [batch, jnp.full((NPAD - n,), G, jnp.int32)]
    ).reshape(NPAD, 1)
    b_row = b.reshape(1, D)
    zeros16 = jnp.zeros((NPAD, 16), jnp.float32)
    zerosd = jnp.zeros((NPAD, D), jnp.float32)

    deg = _make_deg_kernel(ec)(dst_t, zeros16)

    nblk = 8
    br = NPAD // nblk
    y = pl.pallas_call(
        _scale_matmul_kernel,
        grid=(nblk,),
        in_specs=[
            pl.BlockSpec((br, D), lambda i: (i, 0)),
            pl.BlockSpec((d_in, D), lambda i: (0, 0)),
            pl.BlockSpec((NC, br, 16), lambda i: (0, i, 0)),
        ],
        out_specs=pl.BlockSpec((br, D), lambda i: (i, 0)),
        out_shape=jax.ShapeDtypeStruct((NPAD, D), jnp.float32),
    )(x_pad, W, deg)

    p = _make_msg_kernel(ec)(src_t, dst_t, y, zerosd)

    pooled = pl.pallas_call(
        functools.partial(_finish_kernel, nblk, br),
        grid=(nblk,),
        in_specs=[
            pl.BlockSpec((NC, br, D), lambda i: (0, i, 0)),
            pl.BlockSpec((br, D), lambda i: (i, 0)),
            pl.BlockSpec((NC, br, 16), lambda i: (0, i, 0)),
            pl.BlockSpec((1, D), lambda i: (0, 0)),
            pl.BlockSpec((br, 1), lambda i: (i, 0)),
        ],
        out_specs=pl.BlockSpec((G, D), lambda i: (0, 0)),
        out_shape=jax.ShapeDtypeStruct((G, D), jnp.float32),
        scratch_shapes=[pltpu.VMEM((G, D), jnp.float32)],
    )(p, y, deg, b_row, batch_col)

    return pooled
